# baseline (speedup 1.0000x reference)
import jax
import jax.numpy as jnp
import numpy as np
from jax import lax
from jax.experimental import pallas as pl
from jax.experimental.pallas import tpu as pltpu
from jax.experimental.pallas import tpu_sc as plsc

_B, _L, _ACT = 32, 64, 1024
_VOCAB, _EMBED, _TPL = 8192, 256, 4
_BL = _B * _L
_N = _BL * _TPL

_ENC_BM = 512


def _enc_body(x_ref, scale_ref, shift_ref, w_ref, b_ref, z_ref):
    xn = (x_ref[...] - shift_ref[...]) * scale_ref[...]
    z_ref[...] = lax.dot_general(
        xn, w_ref[...], (((1,), (0,)), ((), ())),
        preferred_element_type=jnp.float32) + b_ref[...]


def _encode(x2d, scale, shift, w_perm, b_perm):
    grid = (_BL // _ENC_BM,)
    return pl.pallas_call(
        _enc_body,
        grid=grid,
        in_specs=[
            pl.BlockSpec((_ENC_BM, _ACT), lambda i: (i, 0)),
            pl.BlockSpec((1, _ACT), lambda i: (0, 0)),
            pl.BlockSpec((1, _ACT), lambda i: (0, 0)),
            pl.BlockSpec((_ACT, _EMBED * _TPL), lambda i: (0, 0)),
            pl.BlockSpec((1, _EMBED * _TPL), lambda i: (0, 0)),
        ],
        out_specs=pl.BlockSpec((_ENC_BM, _EMBED * _TPL), lambda i: (i, 0)),
        out_shape=jax.ShapeDtypeStruct((_BL, _EMBED * _TPL), jnp.float32),
        compiler_params=pltpu.CompilerParams(
            dimension_semantics=("parallel",)),
    )(x2d, scale.reshape(1, _ACT), shift.reshape(1, _ACT), w_perm, b_perm)


_DIST_BM = 2048
_DIST_BN = 2048
_BNH = _DIST_BN // 8
_NCHUNK = _VOCAB // _DIST_BN


def _dist_body(z_ref, cb_ref, tok_ref, z2_ref, a_ref, key_ref):
    ci = pl.program_id(1)

    @pl.when(ci == 0)
    def _():
        z = z_ref[...]
        a_ref[...] = jnp.sum(z * z, axis=1, keepdims=True)
        z2_ref[...] = z * 2.0
        key_ref[...] = jnp.full((_DIST_BM, 128), jnp.int32(2**30))

    def half(cb, base):
        csq = lax.dot_general(jnp.ones((8, _EMBED), jnp.float32), cb * cb,
                              (((1,), (1,)), ((), ())),
                              preferred_element_type=jnp.float32)
        m2 = lax.dot_general(z2_ref[...], cb, (((1,), (1,)), ((), ())),
                             preferred_element_type=jnp.float32)
        t1 = a_ref[...] + csq[0:1, :]
        t2 = t1 - m2
        val = t2
        idx = lax.broadcasted_iota(jnp.int32, (_DIST_BM, _BNH), 1) + base
        w = _BNH // 2
        while w >= 128:
            lo_v, hi_v = val[:, :w], val[:, w:]
            lo_i, hi_i = idx[:, :w], idx[:, w:]
            take_hi = hi_v < lo_v
            val = jnp.where(take_hi, hi_v, lo_v)
            idx = jnp.where(take_hi, hi_i, lo_i)
            w //= 2
        d = val - a_ref[...]
        ki = lax.convert_element_type(d * 131072.0, jnp.int32)
        return jnp.bitwise_or(lax.shift_left(ki, 13), idx)

    cb = cb_ref[...]
    keys = [half(cb[s * _BNH:(s + 1) * _BNH, :], ci * _DIST_BN + s * _BNH)
            for s in range(_DIST_BN // _BNH)]
    acc = keys[0]
    for k in keys[1:]:
        acc = jnp.minimum(acc, k)
    key_ref[...] = jnp.minimum(key_ref[...], acc)

    @pl.when(ci == _NCHUNK - 1)
    def _():
        km = jnp.min(key_ref[...], axis=1, keepdims=True)
        tok_ref[...] = jnp.bitwise_and(km, jnp.int32(8191))


def _nearest_tokens(z_flat, codebook):
    grid = (_N // _DIST_BM, _NCHUNK)
    return pl.pallas_call(
        _dist_body,
        grid=grid,
        in_specs=[
            pl.BlockSpec((_DIST_BM, _EMBED), lambda i, j: (i, 0)),
            pl.BlockSpec((_DIST_BN, _EMBED), lambda i, j: (j, 0)),
        ],
        out_specs=pl.BlockSpec((_DIST_BM, 1), lambda i, j: (i, 0)),
        out_shape=jax.ShapeDtypeStruct((_N, 1), jnp.int32),
        scratch_shapes=[
            pltpu.VMEM((_DIST_BM, _EMBED), jnp.float32),
            pltpu.VMEM((_DIST_BM, 1), jnp.float32),
            pltpu.VMEM((_DIST_BM, 128), jnp.int32),
        ],
        compiler_params=pltpu.CompilerParams(
            dimension_semantics=("parallel", "arbitrary")),
    )(z_flat, codebook)


_GATHER_WIN = 128


def _gather_rows(codebook, tokens):
    idx = tokens.reshape(1, _N)
    mesh = plsc.VectorSubcoreMesh(core_axis_name="core",
                                  subcore_axis_name="subcore")

    @pl.kernel(out_type=jax.ShapeDtypeStruct((_N, _EMBED), codebook.dtype),
               mesh=mesh)
    def kern(cb_hbm, i_hbm, o_hbm):
        def body(i_vmem, o_vmem):
            pltpu.sync_copy(cb_hbm.at[i_vmem.at[0]], o_vmem)

        pltpu.emit_pipeline(
            body,
            grid=(_N // _GATHER_WIN,),
            in_specs=[pl.BlockSpec((1, _GATHER_WIN), index_map=lambda i: (0, i))],
            out_specs=[pl.BlockSpec((_GATHER_WIN, _EMBED),
                                    index_map=lambda i: (i, 0))],
            core_axis_name="subcore",
            dimension_semantics=(pltpu.PARALLEL,),
        )(i_hbm, o_hbm)

    return kern(codebook, idx)


_DEC_BM = 512


def _dec_body(zq_ref, scale_ref, shift_ref, w_ref, b_ref, rec_ref):
    r = lax.dot_general(
        zq_ref[...], w_ref[...], (((1,), (0,)), ((), ())),
        preferred_element_type=jnp.float32) + b_ref[...]
    rec_ref[...] = r / scale_ref[...] + shift_ref[...]


def _decode(zq2d, scale, shift, w_perm, dec_b):
    grid = (_BL // _DEC_BM,)
    return pl.pallas_call(
        _dec_body,
        grid=grid,
        in_specs=[
            pl.BlockSpec((_DEC_BM, _EMBED * _TPL), lambda i: (i, 0)),
            pl.BlockSpec((1, _ACT), lambda i: (0, 0)),
            pl.BlockSpec((1, _ACT), lambda i: (0, 0)),
            pl.BlockSpec((_EMBED * _TPL, _ACT), lambda i: (0, 0)),
            pl.BlockSpec((1, _ACT), lambda i: (0, 0)),
        ],
        out_specs=pl.BlockSpec((_DEC_BM, _ACT), lambda i: (i, 0)),
        out_shape=jax.ShapeDtypeStruct((_BL, _ACT), jnp.float32),
        compiler_params=pltpu.CompilerParams(
            dimension_semantics=("parallel",)),
    )(zq2d, scale.reshape(1, _ACT), shift.reshape(1, _ACT), w_perm,
      dec_b.reshape(1, _ACT))


def kernel(x, scale, shift, enc_W, enc_b, dec_W, dec_b, codebook):
    a, v = _EMBED, _TPL
    enc_W_perm = enc_W.reshape(_ACT, a, v).transpose(0, 2, 1).reshape(_ACT, a * v)
    enc_b_perm = enc_b.reshape(a, v).transpose(1, 0).reshape(1, a * v)
    dec_W_perm = dec_W.reshape(a, v, _ACT).transpose(1, 0, 2).reshape(a * v, _ACT)

    x2d = x.reshape(_BL, _ACT)
    z_lin = _encode(x2d, scale, shift, enc_W_perm, enc_b_perm)
    z_flat = z_lin.reshape(_N, a)

    tokens = _nearest_tokens(z_flat, codebook)
    zq_rows = _gather_rows(codebook, tokens.reshape(_N))

    rec2d = _decode(zq_rows.reshape(_BL, a * v), scale, shift,
                    dec_W_perm, dec_b)

    z = z_lin.reshape(_B, _L, v, a).transpose(0, 1, 3, 2)
    z_q = zq_rows.reshape(_B, _L, v, a).transpose(0, 1, 3, 2)
    rec = rec2d.reshape(_B, _L, _ACT)
    return z, z_q, rec


# speedup vs baseline: 1.0754x; 1.0754x over previous
import jax
import jax.numpy as jnp
import numpy as np
from jax import lax
from jax.experimental import pallas as pl
from jax.experimental.pallas import tpu as pltpu
from jax.experimental.pallas import tpu_sc as plsc

_B, _L, _ACT = 32, 64, 1024
_VOCAB, _EMBED, _TPL = 8192, 256, 4
_BL = _B * _L
_N = _BL * _TPL

_ENC_BM = 512


def _enc_body(x_ref, scale_ref, shift_ref, w_ref, b_ref, z_ref):
    xn = (x_ref[...] - shift_ref[...]) * scale_ref[...]
    z_ref[...] = lax.dot_general(
        xn, w_ref[...], (((1,), (0,)), ((), ())),
        preferred_element_type=jnp.float32) + b_ref[...]


def _encode(x2d, scale, shift, w_perm, b_perm):
    grid = (_BL // _ENC_BM,)
    return pl.pallas_call(
        _enc_body,
        grid=grid,
        in_specs=[
            pl.BlockSpec((_ENC_BM, _ACT), lambda i: (i, 0)),
            pl.BlockSpec((1, _ACT), lambda i: (0, 0)),
            pl.BlockSpec((1, _ACT), lambda i: (0, 0)),
            pl.BlockSpec((_ACT, _EMBED * _TPL), lambda i: (0, 0)),
            pl.BlockSpec((1, _EMBED * _TPL), lambda i: (0, 0)),
        ],
        out_specs=pl.BlockSpec((_ENC_BM, _EMBED * _TPL), lambda i: (i, 0)),
        out_shape=jax.ShapeDtypeStruct((_BL, _EMBED * _TPL), jnp.float32),
        compiler_params=pltpu.CompilerParams(
            dimension_semantics=("parallel",)),
    )(x2d, scale.reshape(1, _ACT), shift.reshape(1, _ACT), w_perm, b_perm)


_DIST_BM = 4096
_DIST_BN = 2048
_BNH = _DIST_BN // 4
_NCHUNK = _VOCAB // _DIST_BN


def _dist_body(z_ref, cb_ref, tok_ref, z2_ref, a_ref, key_ref):
    ci = pl.program_id(1)

    @pl.when(ci == 0)
    def _():
        z = z_ref[...]
        a_ref[...] = jnp.sum(z * z, axis=1, keepdims=True)
        z2_ref[...] = z * 2.0
        key_ref[...] = jnp.full((_DIST_BM, 128), jnp.int32(2**30))

    def half(cb, base):
        csq = lax.dot_general(jnp.ones((8, _EMBED), jnp.float32), cb * cb,
                              (((1,), (1,)), ((), ())),
                              preferred_element_type=jnp.float32)
        m2 = lax.dot_general(z2_ref[...], cb, (((1,), (1,)), ((), ())),
                             preferred_element_type=jnp.float32)
        t1 = a_ref[...] + csq[0:1, :]
        t2 = t1 - m2
        val = t2
        idx = lax.broadcasted_iota(jnp.int32, (_DIST_BM, _BNH), 1) + base
        w = _BNH // 2
        while w >= 128:
            lo_v, hi_v = val[:, :w], val[:, w:]
            lo_i, hi_i = idx[:, :w], idx[:, w:]
            take_hi = hi_v < lo_v
            val = jnp.where(take_hi, hi_v, lo_v)
            idx = jnp.where(take_hi, hi_i, lo_i)
            w //= 2
        d = val - a_ref[...]
        ki = lax.convert_element_type(d * 131072.0, jnp.int32)
        return jnp.bitwise_or(lax.shift_left(ki, 13), idx)

    cb = cb_ref[...]
    keys = [half(cb[s * _BNH:(s + 1) * _BNH, :], ci * _DIST_BN + s * _BNH)
            for s in range(_DIST_BN // _BNH)]
    acc = keys[0]
    for k in keys[1:]:
        acc = jnp.minimum(acc, k)
    key_ref[...] = jnp.minimum(key_ref[...], acc)

    @pl.when(ci == _NCHUNK - 1)
    def _():
        km = jnp.min(key_ref[...], axis=1, keepdims=True)
        tok_ref[...] = jnp.bitwise_and(km, jnp.int32(8191))


def _nearest_tokens(z_flat, codebook):
    grid = (_N // _DIST_BM, _NCHUNK)
    return pl.pallas_call(
        _dist_body,
        grid=grid,
        in_specs=[
            pl.BlockSpec((_DIST_BM, _EMBED), lambda i, j: (i, 0)),
            pl.BlockSpec((_DIST_BN, _EMBED), lambda i, j: (j, 0)),
        ],
        out_specs=pl.BlockSpec((_DIST_BM, 1), lambda i, j: (i, 0)),
        out_shape=jax.ShapeDtypeStruct((_N, 1), jnp.int32),
        scratch_shapes=[
            pltpu.VMEM((_DIST_BM, _EMBED), jnp.float32),
            pltpu.VMEM((_DIST_BM, 1), jnp.float32),
            pltpu.VMEM((_DIST_BM, 128), jnp.int32),
        ],
        compiler_params=pltpu.CompilerParams(
            dimension_semantics=("parallel", "arbitrary")),
    )(z_flat, codebook)


_GATHER_WIN = 128


def _gather_rows(codebook, tokens):
    idx = tokens.reshape(1, _N)
    mesh = plsc.VectorSubcoreMesh(core_axis_name="core",
                                  subcore_axis_name="subcore")

    @pl.kernel(out_type=jax.ShapeDtypeStruct((_N, _EMBED), codebook.dtype),
               mesh=mesh)
    def kern(cb_hbm, i_hbm, o_hbm):
        def body(i_vmem, o_vmem):
            pltpu.sync_copy(cb_hbm.at[i_vmem.at[0]], o_vmem)

        pltpu.emit_pipeline(
            body,
            grid=(_N // _GATHER_WIN,),
            in_specs=[pl.BlockSpec((1, _GATHER_WIN), index_map=lambda i: (0, i))],
            out_specs=[pl.BlockSpec((_GATHER_WIN, _EMBED),
                                    index_map=lambda i: (i, 0))],
            core_axis_name="subcore",
            dimension_semantics=(pltpu.PARALLEL,),
        )(i_hbm, o_hbm)

    return kern(codebook, idx)


_DEC_BM = 512


def _dec_body(zq_ref, scale_ref, shift_ref, w_ref, b_ref, rec_ref):
    r = lax.dot_general(
        zq_ref[...], w_ref[...], (((1,), (0,)), ((), ())),
        preferred_element_type=jnp.float32) + b_ref[...]
    rec_ref[...] = r / scale_ref[...] + shift_ref[...]


def _decode(zq2d, scale, shift, w_perm, dec_b):
    grid = (_BL // _DEC_BM,)
    return pl.pallas_call(
        _dec_body,
        grid=grid,
        in_specs=[
            pl.BlockSpec((_DEC_BM, _EMBED * _TPL), lambda i: (i, 0)),
            pl.BlockSpec((1, _ACT), lambda i: (0, 0)),
            pl.BlockSpec((1, _ACT), lambda i: (0, 0)),
            pl.BlockSpec((_EMBED * _TPL, _ACT), lambda i: (0, 0)),
            pl.BlockSpec((1, _ACT), lambda i: (0, 0)),
        ],
        out_specs=pl.BlockSpec((_DEC_BM, _ACT), lambda i: (i, 0)),
        out_shape=jax.ShapeDtypeStruct((_BL, _ACT), jnp.float32),
        compiler_params=pltpu.CompilerParams(
            dimension_semantics=("parallel",)),
    )(zq2d, scale.reshape(1, _ACT), shift.reshape(1, _ACT), w_perm,
      dec_b.reshape(1, _ACT))


def kernel(x, scale, shift, enc_W, enc_b, dec_W, dec_b, codebook):
    a, v = _EMBED, _TPL
    enc_W_perm = enc_W.reshape(_ACT, a, v).transpose(0, 2, 1).reshape(_ACT, a * v)
    enc_b_perm = enc_b.reshape(a, v).transpose(1, 0).reshape(1, a * v)
    dec_W_perm = dec_W.reshape(a, v, _ACT).transpose(1, 0, 2).reshape(a * v, _ACT)

    x2d = x.reshape(_BL, _ACT)
    z_lin = _encode(x2d, scale, shift, enc_W_perm, enc_b_perm)
    z_flat = z_lin.reshape(_N, a)

    tokens = _nearest_tokens(z_flat, codebook)
    zq_rows = _gather_rows(codebook, tokens.reshape(_N))

    rec2d = _decode(zq_rows.reshape(_BL, a * v), scale, shift,
                    dec_W_perm, dec_b)

    z = z_lin.reshape(_B, _L, v, a).transpose(0, 1, 3, 2)
    z_q = zq_rows.reshape(_B, _L, v, a).transpose(0, 1, 3, 2)
    rec = rec2d.reshape(_B, _L, _ACT)
    return z, z_q, rec


# speedup vs baseline: 1.0884x; 1.0120x over previous
import jax
import jax.numpy as jnp
import numpy as np
from jax import lax
from jax.experimental import pallas as pl
from jax.experimental.pallas import tpu as pltpu
from jax.experimental.pallas import tpu_sc as plsc

_B, _L, _ACT = 32, 64, 1024
_VOCAB, _EMBED, _TPL = 8192, 256, 4
_BL = _B * _L
_N = _BL * _TPL

_ENC_BM = 512


def _enc_body(x_ref, scale_ref, shift_ref, w_ref, b_ref, z_ref):
    xn = (x_ref[...] - shift_ref[...]) * scale_ref[...]
    z_ref[...] = lax.dot_general(
        xn, w_ref[...], (((1,), (0,)), ((), ())),
        preferred_element_type=jnp.float32) + b_ref[...]


def _encode(x2d, scale, shift, w_perm, b_perm):
    grid = (_BL // _ENC_BM,)
    return pl.pallas_call(
        _enc_body,
        grid=grid,
        in_specs=[
            pl.BlockSpec((_ENC_BM, _ACT), lambda i: (i, 0)),
            pl.BlockSpec((1, _ACT), lambda i: (0, 0)),
            pl.BlockSpec((1, _ACT), lambda i: (0, 0)),
            pl.BlockSpec((_ACT, _EMBED * _TPL), lambda i: (0, 0)),
            pl.BlockSpec((1, _EMBED * _TPL), lambda i: (0, 0)),
        ],
        out_specs=pl.BlockSpec((_ENC_BM, _EMBED * _TPL), lambda i: (i, 0)),
        out_shape=jax.ShapeDtypeStruct((_BL, _EMBED * _TPL), jnp.float32),
        compiler_params=pltpu.CompilerParams(
            dimension_semantics=("parallel",)),
    )(x2d, scale.reshape(1, _ACT), shift.reshape(1, _ACT), w_perm, b_perm)


_DIST_BM = 8192
_DIST_BN = 2048
_BNH = _DIST_BN // 4
_NCHUNK = _VOCAB // _DIST_BN


def _dist_body(z_ref, cb_ref, tok_ref, z2_ref, a_ref, key_ref):
    ci = pl.program_id(1)

    @pl.when(ci == 0)
    def _():
        z = z_ref[...]
        a_ref[...] = jnp.sum(z * z, axis=1, keepdims=True)
        z2_ref[...] = z * 2.0
        key_ref[...] = jnp.full((_DIST_BM, 128), jnp.int32(2**30))

    def half(cb, base):
        csq = lax.dot_general(jnp.ones((8, _EMBED), jnp.float32), cb * cb,
                              (((1,), (1,)), ((), ())),
                              preferred_element_type=jnp.float32)
        m2 = lax.dot_general(z2_ref[...], cb, (((1,), (1,)), ((), ())),
                             preferred_element_type=jnp.float32)
        t1 = a_ref[...] + csq[0:1, :]
        t2 = t1 - m2
        val = t2
        idx = lax.broadcasted_iota(jnp.int32, (_DIST_BM, _BNH), 1) + base
        w = _BNH // 2
        while w >= 128:
            lo_v, hi_v = val[:, :w], val[:, w:]
            lo_i, hi_i = idx[:, :w], idx[:, w:]
            take_hi = hi_v < lo_v
            val = jnp.where(take_hi, hi_v, lo_v)
            idx = jnp.where(take_hi, hi_i, lo_i)
            w //= 2
        d = val - a_ref[...]
        ki = lax.convert_element_type(d * 131072.0, jnp.int32)
        return jnp.bitwise_or(lax.shift_left(ki, 13), idx)

    cb = cb_ref[...]
    keys = [half(cb[s * _BNH:(s + 1) * _BNH, :], ci * _DIST_BN + s * _BNH)
            for s in range(_DIST_BN // _BNH)]
    acc = keys[0]
    for k in keys[1:]:
        acc = jnp.minimum(acc, k)
    key_ref[...] = jnp.minimum(key_ref[...], acc)

    @pl.when(ci == _NCHUNK - 1)
    def _():
        km = jnp.min(key_ref[...], axis=1, keepdims=True)
        tok_ref[...] = jnp.bitwise_and(km, jnp.int32(8191))


def _nearest_tokens(z_flat, codebook):
    grid = (_N // _DIST_BM, _NCHUNK)
    return pl.pallas_call(
        _dist_body,
        grid=grid,
        in_specs=[
            pl.BlockSpec((_DIST_BM, _EMBED), lambda i, j: (i, 0)),
            pl.BlockSpec((_DIST_BN, _EMBED), lambda i, j: (j, 0)),
        ],
        out_specs=pl.BlockSpec((_DIST_BM, 1), lambda i, j: (i, 0)),
        out_shape=jax.ShapeDtypeStruct((_N, 1), jnp.int32),
        scratch_shapes=[
            pltpu.VMEM((_DIST_BM, _EMBED), jnp.float32),
            pltpu.VMEM((_DIST_BM, 1), jnp.float32),
            pltpu.VMEM((_DIST_BM, 128), jnp.int32),
        ],
        compiler_params=pltpu.CompilerParams(
            dimension_semantics=("parallel", "arbitrary")),
    )(z_flat, codebook)


_GATHER_WIN = 128


def _gather_rows(codebook, tokens):
    idx = tokens.reshape(1, _N)
    mesh = plsc.VectorSubcoreMesh(core_axis_name="core",
                                  subcore_axis_name="subcore")

    @pl.kernel(out_type=jax.ShapeDtypeStruct((_N, _EMBED), codebook.dtype),
               mesh=mesh)
    def kern(cb_hbm, i_hbm, o_hbm):
        def body(i_vmem, o_vmem):
            pltpu.sync_copy(cb_hbm.at[i_vmem.at[0]], o_vmem)

        pltpu.emit_pipeline(
            body,
            grid=(_N // _GATHER_WIN,),
            in_specs=[pl.BlockSpec((1, _GATHER_WIN), index_map=lambda i: (0, i))],
            out_specs=[pl.BlockSpec((_GATHER_WIN, _EMBED),
                                    index_map=lambda i: (i, 0))],
            core_axis_name="subcore",
            dimension_semantics=(pltpu.PARALLEL,),
        )(i_hbm, o_hbm)

    return kern(codebook, idx)


_DEC_BM = 512


def _dec_body(zq_ref, scale_ref, shift_ref, w_ref, b_ref, rec_ref):
    r = lax.dot_general(
        zq_ref[...], w_ref[...], (((1,), (0,)), ((), ())),
        preferred_element_type=jnp.float32) + b_ref[...]
    rec_ref[...] = r / scale_ref[...] + shift_ref[...]


def _decode(zq2d, scale, shift, w_perm, dec_b):
    grid = (_BL // _DEC_BM,)
    return pl.pallas_call(
        _dec_body,
        grid=grid,
        in_specs=[
            pl.BlockSpec((_DEC_BM, _EMBED * _TPL), lambda i: (i, 0)),
            pl.BlockSpec((1, _ACT), lambda i: (0, 0)),
            pl.BlockSpec((1, _ACT), lambda i: (0, 0)),
            pl.BlockSpec((_EMBED * _TPL, _ACT), lambda i: (0, 0)),
            pl.BlockSpec((1, _ACT), lambda i: (0, 0)),
        ],
        out_specs=pl.BlockSpec((_DEC_BM, _ACT), lambda i: (i, 0)),
        out_shape=jax.ShapeDtypeStruct((_BL, _ACT), jnp.float32),
        compiler_params=pltpu.CompilerParams(
            dimension_semantics=("parallel",)),
    )(zq2d, scale.reshape(1, _ACT), shift.reshape(1, _ACT), w_perm,
      dec_b.reshape(1, _ACT))


def kernel(x, scale, shift, enc_W, enc_b, dec_W, dec_b, codebook):
    a, v = _EMBED, _TPL
    enc_W_perm = enc_W.reshape(_ACT, a, v).transpose(0, 2, 1).reshape(_ACT, a * v)
    enc_b_perm = enc_b.reshape(a, v).transpose(1, 0).reshape(1, a * v)
    dec_W_perm = dec_W.reshape(a, v, _ACT).transpose(1, 0, 2).reshape(a * v, _ACT)

    x2d = x.reshape(_BL, _ACT)
    z_lin = _encode(x2d, scale, shift, enc_W_perm, enc_b_perm)
    z_flat = z_lin.reshape(_N, a)

    tokens = _nearest_tokens(z_flat, codebook)
    zq_rows = _gather_rows(codebook, tokens.reshape(_N))

    rec2d = _decode(zq_rows.reshape(_BL, a * v), scale, shift,
                    dec_W_perm, dec_b)

    z = z_lin.reshape(_B, _L, v, a).transpose(0, 1, 3, 2)
    z_q = zq_rows.reshape(_B, _L, v, a).transpose(0, 1, 3, 2)
    rec = rec2d.reshape(_B, _L, _ACT)
    return z, z_q, rec


# speedup vs baseline: 1.1869x; 1.0905x over previous
import jax
import jax.numpy as jnp
import numpy as np
from jax import lax
from jax.experimental import pallas as pl
from jax.experimental.pallas import tpu as pltpu
from jax.experimental.pallas import tpu_sc as plsc

_B, _L, _ACT = 32, 64, 1024
_VOCAB, _EMBED, _TPL = 8192, 256, 4
_BL = _B * _L
_N = _BL * _TPL

_DBN = 2048
_DBS = 512
_NCHUNK = _VOCAB // _DBN

_ONES_BLK = np.zeros((_EMBED * _TPL, _TPL), dtype=np.float32)
for _i in range(_EMBED * _TPL):
    _ONES_BLK[_i, _i // _EMBED] = 1.0


def _encdist_body(x_ref, scale_ref, shift_ref, w_ref, b_ref, ones_ref,
                  cb_ref, z_ref, tok_ref, a4_ref, key_ref):
    ci = pl.program_id(0)

    @pl.when(ci == 0)
    def _():
        xn = (x_ref[...] - shift_ref[...]) * scale_ref[...]
        zl = lax.dot_general(
            xn, w_ref[...], (((1,), (0,)), ((), ())),
            preferred_element_type=jnp.float32) + b_ref[...]
        z_ref[...] = zl
        a4_ref[...] = lax.dot_general(
            zl * zl, ones_ref[...], (((1,), (0,)), ((), ())),
            preferred_element_type=jnp.float32)
        key_ref[...] = jnp.full((_BL, _TPL * 128), jnp.int32(2**30))

    cb = cb_ref[...]
    for s in range(_DBN // _DBS):
        cbs = cb[s * _DBS:(s + 1) * _DBS, :]
        cbs2 = cbs * 2.0
        base = ci * _DBN + s * _DBS
        csq = lax.dot_general(jnp.ones((8, _EMBED), jnp.float32), cbs * cbs,
                              (((1,), (1,)), ((), ())),
                              preferred_element_type=jnp.float32)
        for v in range(_TPL):
            m2 = lax.dot_general(
                z_ref[:, v * _EMBED:(v + 1) * _EMBED], cbs2,
                (((1,), (1,)), ((), ())),
                preferred_element_type=jnp.float32)
            t1 = a4_ref[:, v:v + 1] + csq[0:1, :]
            t2 = t1 - m2
            val = t2
            idx = lax.broadcasted_iota(jnp.int32, (_BL, _DBS), 1) + base
            w = _DBS // 2
            while w >= 128:
                lo_v, hi_v = val[:, :w], val[:, w:]
                lo_i, hi_i = idx[:, :w], idx[:, w:]
                take_hi = hi_v < lo_v
                val = jnp.where(take_hi, hi_v, lo_v)
                idx = jnp.where(take_hi, hi_i, lo_i)
                w //= 2
            d = val - a4_ref[:, v:v + 1]
            ki = lax.convert_element_type(d * 131072.0, jnp.int32)
            key = jnp.bitwise_or(lax.shift_left(ki, 13), idx)
            ks = key_ref[:, v * 128:(v + 1) * 128]
            key_ref[:, v * 128:(v + 1) * 128] = jnp.minimum(ks, key)

    @pl.when(ci == _NCHUNK - 1)
    def _():
        for v in range(_TPL):
            km = jnp.min(key_ref[:, v * 128:(v + 1) * 128], axis=1,
                         keepdims=True)
            tok_ref[:, v:v + 1] = jnp.bitwise_and(km, jnp.int32(8191))


def _encode_and_tokens(x2d, scale, shift, w_perm, b_perm, codebook):
    grid = (_NCHUNK,)
    return pl.pallas_call(
        _encdist_body,
        grid=grid,
        in_specs=[
            pl.BlockSpec((_BL, _ACT), lambda j: (0, 0)),
            pl.BlockSpec((1, _ACT), lambda j: (0, 0)),
            pl.BlockSpec((1, _ACT), lambda j: (0, 0)),
            pl.BlockSpec((_ACT, _EMBED * _TPL), lambda j: (0, 0)),
            pl.BlockSpec((1, _EMBED * _TPL), lambda j: (0, 0)),
            pl.BlockSpec((_EMBED * _TPL, _TPL), lambda j: (0, 0)),
            pl.BlockSpec((_DBN, _EMBED), lambda j: (j, 0)),
        ],
        out_specs=[
            pl.BlockSpec((_BL, _EMBED * _TPL), lambda j: (0, 0)),
            pl.BlockSpec((_BL, _TPL), lambda j: (0, 0)),
        ],
        out_shape=[
            jax.ShapeDtypeStruct((_BL, _EMBED * _TPL), jnp.float32),
            jax.ShapeDtypeStruct((_BL, _TPL), jnp.int32),
        ],
        scratch_shapes=[
            pltpu.VMEM((_BL, _TPL), jnp.float32),
            pltpu.VMEM((_BL, _TPL * 128), jnp.int32),
        ],
        compiler_params=pltpu.CompilerParams(
            dimension_semantics=("arbitrary",)),
    )(x2d, scale.reshape(1, _ACT), shift.reshape(1, _ACT), w_perm, b_perm,
      jnp.asarray(_ONES_BLK), codebook)


_GATHER_WIN = 128


def _gather_rows(codebook, tokens):
    idx = tokens.reshape(1, _N)
    mesh = plsc.VectorSubcoreMesh(core_axis_name="core",
                                  subcore_axis_name="subcore")

    @pl.kernel(out_type=jax.ShapeDtypeStruct((_N, _EMBED), codebook.dtype),
               mesh=mesh)
    def kern(cb_hbm, i_hbm, o_hbm):
        def body(i_vmem, o_vmem):
            pltpu.sync_copy(cb_hbm.at[i_vmem.at[0]], o_vmem)

        pltpu.emit_pipeline(
            body,
            grid=(_N // _GATHER_WIN,),
            in_specs=[pl.BlockSpec((1, _GATHER_WIN), index_map=lambda i: (0, i))],
            out_specs=[pl.BlockSpec((_GATHER_WIN, _EMBED),
                                    index_map=lambda i: (i, 0))],
            core_axis_name="subcore",
            dimension_semantics=(pltpu.PARALLEL,),
        )(i_hbm, o_hbm)

    return kern(codebook, idx)


_DEC_BM = 512


def _dec_body(zq_ref, scale_ref, shift_ref, w_ref, b_ref, rec_ref):
    r = lax.dot_general(
        zq_ref[...], w_ref[...], (((1,), (0,)), ((), ())),
        preferred_element_type=jnp.float32) + b_ref[...]
    rec_ref[...] = r / scale_ref[...] + shift_ref[...]


def _decode(zq2d, scale, shift, w_perm, dec_b):
    grid = (_BL // _DEC_BM,)
    return pl.pallas_call(
        _dec_body,
        grid=grid,
        in_specs=[
            pl.BlockSpec((_DEC_BM, _EMBED * _TPL), lambda i: (i, 0)),
            pl.BlockSpec((1, _ACT), lambda i: (0, 0)),
            pl.BlockSpec((1, _ACT), lambda i: (0, 0)),
            pl.BlockSpec((_EMBED * _TPL, _ACT), lambda i: (0, 0)),
            pl.BlockSpec((1, _ACT), lambda i: (0, 0)),
        ],
        out_specs=pl.BlockSpec((_DEC_BM, _ACT), lambda i: (i, 0)),
        out_shape=jax.ShapeDtypeStruct((_BL, _ACT), jnp.float32),
        compiler_params=pltpu.CompilerParams(
            dimension_semantics=("parallel",)),
    )(zq2d, scale.reshape(1, _ACT), shift.reshape(1, _ACT), w_perm,
      dec_b.reshape(1, _ACT))


def kernel(x, scale, shift, enc_W, enc_b, dec_W, dec_b, codebook):
    a, v = _EMBED, _TPL
    enc_W_perm = enc_W.reshape(_ACT, a, v).transpose(0, 2, 1).reshape(_ACT, a * v)
    enc_b_perm = enc_b.reshape(a, v).transpose(1, 0).reshape(1, a * v)
    dec_W_perm = dec_W.reshape(a, v, _ACT).transpose(1, 0, 2).reshape(a * v, _ACT)

    x2d = x.reshape(_BL, _ACT)
    z_lin, tok2d = _encode_and_tokens(x2d, scale, shift, enc_W_perm,
                                      enc_b_perm, codebook)
    zq_rows = _gather_rows(codebook, tok2d.reshape(_N))

    rec2d = _decode(zq_rows.reshape(_BL, a * v), scale, shift,
                    dec_W_perm, dec_b)

    z = z_lin.reshape(_B, _L, v, a).transpose(0, 1, 3, 2)
    z_q = zq_rows.reshape(_B, _L, v, a).transpose(0, 1, 3, 2)
    rec = rec2d.reshape(_B, _L, _ACT)
    return z, z_q, rec
